# Initial kernel scaffold; baseline (speedup 1.0000x reference)
import jax
import jax.numpy as jnp
from jax.experimental import pallas as pl


def kernel(x, edge_index, W_gate, b_gate, W_conv, b_conv):
    raise NotImplementedError("write your pallas kernel here")


# speedup vs baseline: 22.4201x; 22.4201x over previous
import functools

import jax
import jax.numpy as jnp
from jax import lax
from jax.experimental import pallas as pl
from jax.experimental.pallas import tpu as pltpu
from jax.experimental.pallas import tpu_sc as plsc

_N = 10000
_E = 160000
_H = 256
_GATE = 7
_GH = 512
_PG = 128

_NT = 16
_NP = 10240
_EC = _E // _NT
_NPT = _NP // _NT
_L = 16


def _rsqrt16(d):
    i = plsc.bitcast(d, jnp.int32)
    i = jnp.int32(0x5F3759DF) - (i >> 1)
    y = plsc.bitcast(i, jnp.float32)
    for _ in range(3):
        y = y * (jnp.float32(1.5) - jnp.float32(0.5) * d * y * y)
    return y


def _edge_body(src_hbm, dst_hbm, coef_hbm,
               src_v, dst_v, w_v, loc_v,
               indeg_sh, outdeg_sh, c_sh):
    tid = lax.axis_index("s")
    ebase = tid * _EC
    nbase = tid * _NPT

    pltpu.sync_copy(src_hbm.at[pl.ds(ebase, _EC)], src_v)
    pltpu.sync_copy(dst_hbm.at[pl.ds(ebase, _EC)], dst_v)

    def zero_chunk(i, _):
        loc_v[pl.ds(i * _L, _L)] = jnp.zeros((_L,), jnp.float32)
        return 0
    lax.fori_loop(0, _NPT // _L, zero_chunk, 0)
    pltpu.sync_copy(loc_v.at[pl.ds(0, _NPT)], indeg_sh.at[pl.ds(nbase, _NPT)])
    pltpu.sync_copy(loc_v.at[pl.ds(0, _NPT)], outdeg_sh.at[pl.ds(nbase, _NPT)])
    pltpu.sync_copy(loc_v.at[pl.ds(0, _NPT)], c_sh.at[pl.ds(nbase, _NPT)])

    def ones_chunk(i, _):
        w_v[pl.ds(i * _L, _L)] = jnp.ones((_L,), jnp.float32)
        return 0
    lax.fori_loop(0, _EC // _L, ones_chunk, 0)
    plsc.subcore_barrier()

    pltpu.sync_copy(w_v, indeg_sh.at[dst_v], add=True)
    pltpu.sync_copy(w_v, outdeg_sh.at[src_v], add=True)
    plsc.subcore_barrier()

    pltpu.sync_copy(indeg_sh.at[pl.ds(nbase, _NPT)], loc_v.at[pl.ds(0, _NPT)])

    def rinv_chunk(i, _):
        d = loc_v[pl.ds(i * _L, _L)]
        loc_v[pl.ds(i * _L, _L)] = _rsqrt16(jnp.maximum(d, 1.0))
        return 0
    lax.fori_loop(0, _NPT // _L, rinv_chunk, 0)
    plsc.subcore_barrier()
    pltpu.sync_copy(loc_v.at[pl.ds(0, _NPT)], indeg_sh.at[pl.ds(nbase, _NPT)])
    plsc.subcore_barrier()

    pltpu.sync_copy(indeg_sh, loc_v.at[pl.ds(0, _NP)])

    def gather_chunk(i, _):
        idx = dst_v[pl.ds(i * _L, _L)]
        w_v[pl.ds(i * _L, _L)] = plsc.load_gather(loc_v, [idx])
        return 0
    lax.fori_loop(0, _EC // _L, gather_chunk, 0)

    pltpu.sync_copy(w_v, c_sh.at[src_v], add=True)
    plsc.subcore_barrier()

    pltpu.sync_copy(outdeg_sh.at[pl.ds(nbase, _NPT)], loc_v.at[pl.ds(0, _NPT)])
    pltpu.sync_copy(c_sh.at[pl.ds(nbase, _NPT)], loc_v.at[pl.ds(_NPT, _NPT)])

    def coef_chunk(i, _):
        od = loc_v[pl.ds(i * _L, _L)]
        cv = loc_v[pl.ds(_NPT + i * _L, _L)]
        loc_v[pl.ds(i * _L, _L)] = _rsqrt16(jnp.maximum(od, 1.0)) * cv
        return 0
    lax.fori_loop(0, _NPT // _L, coef_chunk, 0)
    pltpu.sync_copy(loc_v.at[pl.ds(0, _NPT)], coef_hbm.at[pl.ds(nbase, _NPT)])


_edge_kernel = functools.partial(
    pl.kernel,
    mesh=plsc.VectorSubcoreMesh(
        core_axis_name="c", subcore_axis_name="s", num_cores=1),
    compiler_params=pltpu.CompilerParams(needs_layout_passes=False),
    out_type=jax.ShapeDtypeStruct((_NP,), jnp.float32),
    scratch_types=[
        pltpu.VMEM((_EC,), jnp.int32),
        pltpu.VMEM((_EC,), jnp.int32),
        pltpu.VMEM((_EC,), jnp.float32),
        pltpu.VMEM((2 * _NPT + _NP,), jnp.float32),
        pltpu.VMEM_SHARED((_NP,), jnp.float32),
        pltpu.VMEM_SHARED((_NP,), jnp.float32),
        pltpu.VMEM_SHARED((_NP,), jnp.float32),
    ],
)(_edge_body)


_BM = 2000


def _dense_body(x_ref, coef_ref, wg_ref, bg_ref, wc_ref, bc_ref,
                out_ref, acc_ref):
    i = pl.program_id(0)

    @pl.when(i == 0)
    def _():
        acc_ref[...] = jnp.zeros_like(acc_ref)

    z = jnp.dot(x_ref[...], wg_ref[...],
                preferred_element_type=jnp.float32) + bg_ref[...]
    h = jax.nn.sigmoid(z)
    acc_ref[...] += jnp.sum(h * coef_ref[...], axis=0, keepdims=True)

    @pl.when(i == pl.num_programs(0) - 1)
    def _():
        s = acc_ref[...] * jnp.float32(1.0 / _N)
        out_ref[...] = jnp.dot(s, wc_ref[...],
                               preferred_element_type=jnp.float32) + bc_ref[...]


def _dense_call(x, coef, wg, bg, wc, bc):
    return pl.pallas_call(
        _dense_body,
        grid=(_N // _BM,),
        in_specs=[
            pl.BlockSpec((_BM, _H), lambda i: (i, 0)),
            pl.BlockSpec((_BM, 1), lambda i: (i, 0)),
            pl.BlockSpec((_H, _PG), lambda i: (0, 0)),
            pl.BlockSpec((1, _PG), lambda i: (0, 0)),
            pl.BlockSpec((_PG, _GH), lambda i: (0, 0)),
            pl.BlockSpec((1, _GH), lambda i: (0, 0)),
        ],
        out_specs=pl.BlockSpec((1, _GH), lambda i: (0, 0)),
        out_shape=jax.ShapeDtypeStruct((1, _GH), jnp.float32),
        scratch_shapes=[pltpu.VMEM((1, _PG), jnp.float32)],
    )(x, coef, wg, bg, wc, bc)


def kernel(x, edge_index, W_gate, b_gate, W_conv, b_conv):
    src = edge_index[0]
    dst = edge_index[1]
    coef = _edge_kernel(src, dst)[:_N].reshape(_N, 1)
    wg = jnp.zeros((_H, _PG), jnp.float32).at[:, :_GATE].set(W_gate)
    bg = jnp.zeros((1, _PG), jnp.float32).at[0, :_GATE].set(b_gate)
    wc = jnp.zeros((_PG, _GH), jnp.float32).at[:_GATE, :].set(W_conv)
    bc = b_conv.reshape(1, _GH)
    return _dense_call(x, coef, wg, bg, wc, bc)


# speedup vs baseline: 28.4926x; 1.2709x over previous
import functools

import jax
import jax.numpy as jnp
from jax import lax
from jax.experimental import pallas as pl
from jax.experimental.pallas import tpu as pltpu
from jax.experimental.pallas import tpu_sc as plsc

_N = 10000
_E = 160000
_H = 256
_GATE = 7
_GH = 512

_NT = 16
_NP = 10240
_EC = _E // _NT
_NPT = _NP // _NT
_L = 16


def _rsqrt16(d):
    i = plsc.bitcast(d, jnp.int32)
    i = jnp.int32(0x5F3759DF) - (i >> 1)
    y = plsc.bitcast(i, jnp.float32)
    for _ in range(3):
        y = y * (jnp.float32(1.5) - jnp.float32(0.5) * d * y * y)
    return y


def _edge_body(src_hbm, dst_hbm, c_hbm, od_hbm,
               src_v, dst_v, w_v, loc_v, tab_v,
               indeg_sh, outdeg_sh, c_sh):
    tid = lax.axis_index("s")
    ebase = tid * _EC
    nbase = tid * _NPT

    pltpu.sync_copy(src_hbm.at[pl.ds(ebase, _EC)], src_v)
    pltpu.sync_copy(dst_hbm.at[pl.ds(ebase, _EC)], dst_v)

    @plsc.parallel_loop(0, _NPT // _L, unroll=8)
    def _(i):
        loc_v[pl.ds(i * _L, _L)] = jnp.zeros((_L,), jnp.float32)

    pltpu.sync_copy(loc_v.at[pl.ds(0, _NPT)], indeg_sh.at[pl.ds(nbase, _NPT)])
    pltpu.sync_copy(loc_v.at[pl.ds(0, _NPT)], outdeg_sh.at[pl.ds(nbase, _NPT)])
    pltpu.sync_copy(loc_v.at[pl.ds(0, _NPT)], c_sh.at[pl.ds(nbase, _NPT)])

    @plsc.parallel_loop(0, _EC // _L, unroll=5)
    def _(i):
        w_v[pl.ds(i * _L, _L)] = jnp.ones((_L,), jnp.float32)

    plsc.subcore_barrier()

    pltpu.sync_copy(w_v, indeg_sh.at[dst_v], add=True)
    pltpu.sync_copy(w_v, outdeg_sh.at[src_v], add=True)
    plsc.subcore_barrier()

    pltpu.sync_copy(outdeg_sh.at[pl.ds(nbase, _NPT)],
                    od_hbm.at[0, pl.ds(nbase, _NPT)])

    pltpu.sync_copy(indeg_sh.at[pl.ds(nbase, _NPT)], loc_v.at[pl.ds(0, _NPT)])

    @plsc.parallel_loop(0, _NPT // _L, unroll=8)
    def _(i):
        d = loc_v[pl.ds(i * _L, _L)]
        loc_v[pl.ds(i * _L, _L)] = _rsqrt16(jnp.maximum(d, 1.0))

    pltpu.sync_copy(loc_v.at[pl.ds(0, _NPT)], indeg_sh.at[pl.ds(nbase, _NPT)])
    plsc.subcore_barrier()

    pltpu.sync_copy(indeg_sh, tab_v)

    @plsc.parallel_loop(0, _EC // _L, unroll=5)
    def _(i):
        idx = dst_v[pl.ds(i * _L, _L)]
        w_v[pl.ds(i * _L, _L)] = plsc.load_gather(tab_v, [idx])

    pltpu.sync_copy(w_v, c_sh.at[src_v], add=True)
    plsc.subcore_barrier()

    pltpu.sync_copy(c_sh.at[pl.ds(nbase, _NPT)],
                    c_hbm.at[0, pl.ds(nbase, _NPT)])


_edge_kernel = functools.partial(
    pl.kernel,
    mesh=plsc.VectorSubcoreMesh(
        core_axis_name="c", subcore_axis_name="s", num_cores=1),
    compiler_params=pltpu.CompilerParams(needs_layout_passes=False),
    out_type=[jax.ShapeDtypeStruct((1, _NP), jnp.float32),
              jax.ShapeDtypeStruct((1, _NP), jnp.float32)],
    scratch_types=[
        pltpu.VMEM((_EC,), jnp.int32),
        pltpu.VMEM((_EC,), jnp.int32),
        pltpu.VMEM((_EC,), jnp.float32),
        pltpu.VMEM((_NPT,), jnp.float32),
        pltpu.VMEM((_NP,), jnp.float32),
        pltpu.VMEM_SHARED((_NP,), jnp.float32),
        pltpu.VMEM_SHARED((_NP,), jnp.float32),
        pltpu.VMEM_SHARED((_NP,), jnp.float32),
    ],
)(_edge_body)


_BM = 2048


def _dense_body(x_ref, c_ref, od_ref, wg_ref, bg_ref, wc_ref, bc_ref,
                out_ref, acc_ref):
    i = pl.program_id(0)

    @pl.when(i == 0)
    def _():
        acc_ref[...] = jnp.zeros_like(acc_ref)

    coef = lax.rsqrt(jnp.maximum(od_ref[...], 1.0)) * c_ref[...]
    z = jnp.dot(x_ref[...], wg_ref[...],
                preferred_element_type=jnp.float32) + bg_ref[...]
    h = jax.nn.sigmoid(z)
    row = lax.broadcasted_iota(jnp.int32, (_BM, 1), 0) + i * _BM
    h = jnp.where(row < _N, h, 0.0)
    acc_ref[...] += jnp.dot(coef, h, preferred_element_type=jnp.float32)

    @pl.when(i == pl.num_programs(0) - 1)
    def _():
        s = acc_ref[...] * jnp.float32(1.0 / _N)
        out_ref[...] = jnp.dot(s, wc_ref[...],
                               preferred_element_type=jnp.float32) + bc_ref[...]


def _dense_call(x, c, od, wg, bg, wc, bc):
    return pl.pallas_call(
        _dense_body,
        grid=(_NP // _BM,),
        in_specs=[
            pl.BlockSpec((_BM, _H), lambda i: (i, 0)),
            pl.BlockSpec((1, _BM), lambda i: (0, i)),
            pl.BlockSpec((1, _BM), lambda i: (0, i)),
            pl.BlockSpec((_H, _GATE), lambda i: (0, 0)),
            pl.BlockSpec((1, _GATE), lambda i: (0, 0)),
            pl.BlockSpec((_GATE, _GH), lambda i: (0, 0)),
            pl.BlockSpec((1, _GH), lambda i: (0, 0)),
        ],
        out_specs=pl.BlockSpec((1, _GH), lambda i: (0, 0)),
        out_shape=jax.ShapeDtypeStruct((1, _GH), jnp.float32),
        scratch_shapes=[pltpu.VMEM((1, _GATE), jnp.float32)],
    )(x, c, od, wg, bg, wc, bc)


def kernel(x, edge_index, W_gate, b_gate, W_conv, b_conv):
    c_row, od_row = _edge_kernel(edge_index[0], edge_index[1])
    return _dense_call(x, c_row, od_row,
                       W_gate, b_gate.reshape(1, _GATE),
                       W_conv, b_conv.reshape(1, _GH))


# speedup vs baseline: 31.8857x; 1.1191x over previous
import functools

import jax
import jax.numpy as jnp
from jax import lax
from jax.experimental import pallas as pl
from jax.experimental.pallas import tpu as pltpu
from jax.experimental.pallas import tpu_sc as plsc

_N = 10000
_E = 160000
_H = 256
_GATE = 7
_GH = 512

_NT = 16
_NP = 10240
_ECA = 10240
_ECT = _E - 15 * _ECA
_NPT = _NP // _NT
_L = 16


def _rsqrt16(d):
    i = plsc.bitcast(d, jnp.int32)
    i = jnp.int32(0x5F3759DF) - (i >> 1)
    y = plsc.bitcast(i, jnp.float32)
    for _ in range(3):
        y = y * (jnp.float32(1.5) - jnp.float32(0.5) * d * y * y)
    return y


def _edge_body(ei_hbm, c_hbm, od_hbm,
               ei_v, src_v, dst_v, w_v, loc_v, tab_v,
               indeg_sh, outdeg_sh, c_sh):
    tid = lax.axis_index("s")
    nbase = tid * _NPT

    @pl.when(tid < _NT - 1)
    def _():
        pltpu.sync_copy(ei_hbm.at[:, pl.ds(tid * _ECA, _ECA)], ei_v)

    @pl.when(tid == _NT - 1)
    def _():
        pltpu.sync_copy(ei_hbm.at[:, pl.ds((_NT - 1) * _ECA, _ECT)],
                        ei_v.at[:, pl.ds(0, _ECT)])

    @plsc.parallel_loop(0, _ECA // _L, unroll=4)
    def _(i):
        src_v[pl.ds(i * _L, _L)] = ei_v[0, pl.ds(i * _L, _L)]
        dst_v[pl.ds(i * _L, _L)] = ei_v[1, pl.ds(i * _L, _L)]

    @pl.when(tid == _NT - 1)
    def _():
        @plsc.parallel_loop(0, (_ECA - _ECT) // _L, unroll=8)
        def _(i):
            src_v[pl.ds(_ECT + i * _L, _L)] = jnp.full((_L,), -1, jnp.int32)
            dst_v[pl.ds(_ECT + i * _L, _L)] = jnp.full((_L,), -1, jnp.int32)

    src_idx = plsc.Indices(src_v, ignored_value=-1)
    dst_idx = plsc.Indices(dst_v, ignored_value=-1)

    @plsc.parallel_loop(0, _NPT // _L, unroll=8)
    def _(i):
        loc_v[pl.ds(i * _L, _L)] = jnp.zeros((_L,), jnp.float32)

    pltpu.sync_copy(loc_v.at[pl.ds(0, _NPT)], indeg_sh.at[pl.ds(nbase, _NPT)])
    pltpu.sync_copy(loc_v.at[pl.ds(0, _NPT)], outdeg_sh.at[pl.ds(nbase, _NPT)])
    pltpu.sync_copy(loc_v.at[pl.ds(0, _NPT)], c_sh.at[pl.ds(nbase, _NPT)])

    @plsc.parallel_loop(0, _ECA // _L, unroll=5)
    def _(i):
        w_v[pl.ds(i * _L, _L)] = jnp.ones((_L,), jnp.float32)

    plsc.subcore_barrier()

    pltpu.sync_copy(w_v, indeg_sh.at[dst_idx], add=True)
    pltpu.sync_copy(w_v, outdeg_sh.at[src_idx], add=True)
    plsc.subcore_barrier()

    pltpu.sync_copy(outdeg_sh.at[pl.ds(nbase, _NPT)],
                    od_hbm.at[0, pl.ds(nbase, _NPT)])

    pltpu.sync_copy(indeg_sh.at[pl.ds(nbase, _NPT)], loc_v.at[pl.ds(0, _NPT)])

    @plsc.parallel_loop(0, _NPT // _L, unroll=8)
    def _(i):
        d = loc_v[pl.ds(i * _L, _L)]
        loc_v[pl.ds(i * _L, _L)] = _rsqrt16(jnp.maximum(d, 1.0))

    pltpu.sync_copy(loc_v.at[pl.ds(0, _NPT)], indeg_sh.at[pl.ds(nbase, _NPT)])
    plsc.subcore_barrier()

    pltpu.sync_copy(indeg_sh, tab_v)

    @plsc.parallel_loop(0, _ECA // _L, unroll=5)
    def _(i):
        idx = jnp.maximum(dst_v[pl.ds(i * _L, _L)], 0)
        w_v[pl.ds(i * _L, _L)] = plsc.load_gather(tab_v, [idx])

    pltpu.sync_copy(w_v, c_sh.at[src_idx], add=True)
    plsc.subcore_barrier()

    pltpu.sync_copy(c_sh.at[pl.ds(nbase, _NPT)],
                    c_hbm.at[0, pl.ds(nbase, _NPT)])


_edge_kernel = functools.partial(
    pl.kernel,
    mesh=plsc.VectorSubcoreMesh(
        core_axis_name="c", subcore_axis_name="s", num_cores=1),
    compiler_params=pltpu.CompilerParams(needs_layout_passes=False),
    out_type=[jax.ShapeDtypeStruct((1, _NP), jnp.float32),
              jax.ShapeDtypeStruct((1, _NP), jnp.float32)],
    scratch_types=[
        pltpu.VMEM((2, _ECA), jnp.int32),
        pltpu.VMEM((_ECA,), jnp.int32),
        pltpu.VMEM((_ECA,), jnp.int32),
        pltpu.VMEM((_ECA,), jnp.float32),
        pltpu.VMEM((_NPT,), jnp.float32),
        pltpu.VMEM((_NP,), jnp.float32),
        pltpu.VMEM_SHARED((_NP,), jnp.float32),
        pltpu.VMEM_SHARED((_NP,), jnp.float32),
        pltpu.VMEM_SHARED((_NP,), jnp.float32),
    ],
)(_edge_body)


_BM = 2048


def _dense_body(x_ref, c_ref, od_ref, wg_ref, bg_ref, wc_ref, bc_ref,
                out_ref, acc_ref):
    i = pl.program_id(0)

    @pl.when(i == 0)
    def _():
        acc_ref[...] = jnp.zeros_like(acc_ref)

    coef = lax.rsqrt(jnp.maximum(od_ref[...], 1.0)) * c_ref[...]
    z = jnp.dot(x_ref[...], wg_ref[...],
                preferred_element_type=jnp.float32) + bg_ref[...]
    h = jax.nn.sigmoid(z)
    row = lax.broadcasted_iota(jnp.int32, (_BM, 1), 0) + i * _BM
    h = jnp.where(row < _N, h, 0.0)
    acc_ref[...] += jnp.dot(coef, h, preferred_element_type=jnp.float32)

    @pl.when(i == pl.num_programs(0) - 1)
    def _():
        s = acc_ref[...] * jnp.float32(1.0 / _N)
        out_ref[...] = jnp.dot(s, wc_ref[...],
                               preferred_element_type=jnp.float32) + bc_ref[...]


def _dense_call(x, c, od, wg, bg, wc, bc):
    return pl.pallas_call(
        _dense_body,
        grid=(_NP // _BM,),
        in_specs=[
            pl.BlockSpec((_BM, _H), lambda i: (i, 0)),
            pl.BlockSpec((1, _BM), lambda i: (0, i)),
            pl.BlockSpec((1, _BM), lambda i: (0, i)),
            pl.BlockSpec((_H, _GATE), lambda i: (0, 0)),
            pl.BlockSpec((1, _GATE), lambda i: (0, 0)),
            pl.BlockSpec((_GATE, _GH), lambda i: (0, 0)),
            pl.BlockSpec((1, _GH), lambda i: (0, 0)),
        ],
        out_specs=pl.BlockSpec((1, _GH), lambda i: (0, 0)),
        out_shape=jax.ShapeDtypeStruct((1, _GH), jnp.float32),
        scratch_shapes=[pltpu.VMEM((1, _GATE), jnp.float32)],
    )(x, c, od, wg, bg, wc, bc)


def kernel(x, edge_index, W_gate, b_gate, W_conv, b_conv):
    c_row, od_row = _edge_kernel(edge_index)
    return _dense_call(x, c_row, od_row,
                       W_gate, b_gate.reshape(1, _GATE),
                       W_conv, b_conv.reshape(1, _GH))


# speedup vs baseline: 36.5204x; 1.1454x over previous
import functools

import jax
import jax.numpy as jnp
from jax import lax
from jax.experimental import pallas as pl
from jax.experimental.pallas import tpu as pltpu
from jax.experimental.pallas import tpu_sc as plsc

_N = 10000
_E = 160000
_H = 256
_GATE = 7
_GH = 512

_NT = 16
_NP = 10240
_ECA = 10240
_ECT = _E - 15 * _ECA
_NPT = _NP // _NT
_L = 16


def _rsqrt16(d):
    i = plsc.bitcast(d, jnp.int32)
    i = jnp.int32(0x5F3759DF) - (i >> 1)
    y = plsc.bitcast(i, jnp.float32)
    for _ in range(3):
        y = y * (jnp.float32(1.5) - jnp.float32(0.5) * d * y * y)
    return y


def _edge_body(ei_hbm, c_hbm, od_hbm,
               ei_v, src_v, dst_v, w_v, loc_v, tab_v,
               indeg_sh, outdeg_sh, c_sh):
    tid = lax.axis_index("s")
    nbase = tid * _NPT

    @pl.when(tid < _NT - 1)
    def _():
        pltpu.sync_copy(ei_hbm.at[:, pl.ds(tid * _ECA, _ECA)], ei_v)

    @pl.when(tid == _NT - 1)
    def _():
        pltpu.sync_copy(ei_hbm.at[:, pl.ds((_NT - 1) * _ECA, _ECT)],
                        ei_v.at[:, pl.ds(0, _ECT)])

    @plsc.parallel_loop(0, _ECA // _L, unroll=4)
    def _(i):
        src_v[pl.ds(i * _L, _L)] = ei_v[0, pl.ds(i * _L, _L)]
        dst_v[pl.ds(i * _L, _L)] = ei_v[1, pl.ds(i * _L, _L)]

    @pl.when(tid == _NT - 1)
    def _():
        @plsc.parallel_loop(0, (_ECA - _ECT) // _L, unroll=8)
        def _(i):
            src_v[pl.ds(_ECT + i * _L, _L)] = jnp.full((_L,), -1, jnp.int32)
            dst_v[pl.ds(_ECT + i * _L, _L)] = jnp.full((_L,), -1, jnp.int32)

    src_idx = plsc.Indices(src_v, ignored_value=-1)
    dst_idx = plsc.Indices(dst_v, ignored_value=-1)

    @plsc.parallel_loop(0, _NPT // _L, unroll=8)
    def _(i):
        loc_v[pl.ds(i * _L, _L)] = jnp.zeros((_L,), jnp.float32)

    pltpu.sync_copy(loc_v.at[pl.ds(0, _NPT)], indeg_sh.at[pl.ds(nbase, _NPT)])
    pltpu.sync_copy(loc_v.at[pl.ds(0, _NPT)], outdeg_sh.at[pl.ds(nbase, _NPT)])
    pltpu.sync_copy(loc_v.at[pl.ds(0, _NPT)], c_sh.at[pl.ds(nbase, _NPT)])

    @plsc.parallel_loop(0, _ECA // _L, unroll=5)
    def _(i):
        w_v[pl.ds(i * _L, _L)] = jnp.ones((_L,), jnp.float32)

    plsc.subcore_barrier()

    pltpu.sync_copy(w_v, indeg_sh.at[dst_idx], add=True)
    pltpu.sync_copy(w_v, outdeg_sh.at[src_idx], add=True)
    plsc.subcore_barrier()

    pltpu.sync_copy(outdeg_sh.at[pl.ds(nbase, _NPT)],
                    od_hbm.at[0, pl.ds(nbase, _NPT)])

    pltpu.sync_copy(indeg_sh.at[pl.ds(nbase, _NPT)], loc_v.at[pl.ds(0, _NPT)])

    @plsc.parallel_loop(0, _NPT // _L, unroll=8)
    def _(i):
        d = loc_v[pl.ds(i * _L, _L)]
        loc_v[pl.ds(i * _L, _L)] = _rsqrt16(jnp.maximum(d, 1.0))

    pltpu.sync_copy(loc_v.at[pl.ds(0, _NPT)], indeg_sh.at[pl.ds(nbase, _NPT)])
    plsc.subcore_barrier()

    pltpu.sync_copy(indeg_sh, tab_v)

    @plsc.parallel_loop(0, _ECA // _L, unroll=5)
    def _(i):
        idx = jnp.maximum(dst_v[pl.ds(i * _L, _L)], 0)
        w_v[pl.ds(i * _L, _L)] = plsc.load_gather(tab_v, [idx])

    pltpu.sync_copy(w_v, c_sh.at[src_idx], add=True)
    plsc.subcore_barrier()

    pltpu.sync_copy(c_sh.at[pl.ds(nbase, _NPT)],
                    c_hbm.at[0, pl.ds(nbase, _NPT)])


_edge_kernel = functools.partial(
    pl.kernel,
    mesh=plsc.VectorSubcoreMesh(
        core_axis_name="c", subcore_axis_name="s", num_cores=1),
    compiler_params=pltpu.CompilerParams(needs_layout_passes=False),
    out_type=[jax.ShapeDtypeStruct((1, _NP), jnp.float32),
              jax.ShapeDtypeStruct((1, _NP), jnp.float32)],
    scratch_types=[
        pltpu.VMEM((2, _ECA), jnp.int32),
        pltpu.VMEM((_ECA,), jnp.int32),
        pltpu.VMEM((_ECA,), jnp.int32),
        pltpu.VMEM((_ECA,), jnp.float32),
        pltpu.VMEM((_NPT,), jnp.float32),
        pltpu.VMEM((_NP,), jnp.float32),
        pltpu.VMEM_SHARED((_NP,), jnp.float32),
        pltpu.VMEM_SHARED((_NP,), jnp.float32),
        pltpu.VMEM_SHARED((_NP,), jnp.float32),
    ],
)(_edge_body)


_BM = 2048


def _h_body(x_ref, wg_ref, bg_ref, ht_ref):
    i = pl.program_id(0)
    z = jnp.dot(x_ref[...], wg_ref[...],
                preferred_element_type=jnp.float32) + bg_ref[...]
    h = jax.nn.sigmoid(z)
    row = lax.broadcasted_iota(jnp.int32, (_BM, 1), 0) + i * _BM
    h = jnp.where(row < _N, h, 0.0)
    ht_ref[...] = jnp.swapaxes(h, 0, 1)


def _h_call(x, wg, bg):
    return pl.pallas_call(
        _h_body,
        grid=(_NP // _BM,),
        in_specs=[
            pl.BlockSpec((_BM, _H), lambda i: (i, 0)),
            pl.BlockSpec((_H, _GATE), lambda i: (0, 0)),
            pl.BlockSpec((1, _GATE), lambda i: (0, 0)),
        ],
        out_specs=pl.BlockSpec((_GATE, _BM), lambda i: (0, i)),
        out_shape=jax.ShapeDtypeStruct((_GATE, _NP), jnp.float32),
    )(x, wg, bg)


def _final_body(ht_ref, c_ref, od_ref, wc_ref, bc_ref, out_ref):
    coef = lax.rsqrt(jnp.maximum(od_ref[...], 1.0)) * c_ref[...]
    p = ht_ref[...] * coef
    s = jnp.sum(p, axis=1, keepdims=True) * jnp.float32(1.0 / _N)
    acc = bc_ref[...]
    for j in range(_GATE):
        acc = acc + s[j:j + 1, 0:1] * wc_ref[j:j + 1, :]
    out_ref[...] = acc


def _final_call(ht, c, od, wc, bc):
    return pl.pallas_call(
        _final_body,
        in_specs=[
            pl.BlockSpec((_GATE, _NP), lambda: (0, 0)),
            pl.BlockSpec((1, _NP), lambda: (0, 0)),
            pl.BlockSpec((1, _NP), lambda: (0, 0)),
            pl.BlockSpec((_GATE, _GH), lambda: (0, 0)),
            pl.BlockSpec((1, _GH), lambda: (0, 0)),
        ],
        out_specs=pl.BlockSpec((1, _GH), lambda: (0, 0)),
        out_shape=jax.ShapeDtypeStruct((1, _GH), jnp.float32),
    )(ht, c, od, wc, bc)


def kernel(x, edge_index, W_gate, b_gate, W_conv, b_conv):
    ht = _h_call(x, W_gate, b_gate.reshape(1, _GATE))
    c_row, od_row = _edge_kernel(edge_index)
    return _final_call(ht, c_row, od_row, W_conv, b_conv.reshape(1, _GH))
